# baseline (speedup 1.0000x reference)
import jax
import jax.numpy as jnp
from jax import lax
from jax.experimental import pallas as pl
from jax.experimental.pallas import tpu as pltpu
from jax.experimental.pallas import tpu_sc as plsc

_ROWS = 100000
_COLS = 512
_NWORKERS = 32
_SHARD = 3128
_CHUNK = 112
_NCH = 28
_LAST_BASE = _ROWS - _SHARD


def _chunk_off(j):
    return j * _CHUNK if j < _NCH - 1 else _SHARD - _CHUNK


_mesh = plsc.VectorSubcoreMesh(core_axis_name="c", subcore_axis_name="s")


@pl.kernel(
    mesh=_mesh,
    out_type=jax.ShapeDtypeStruct((_ROWS, _COLS), jnp.float32),
    scratch_types=[
        pltpu.VMEM((2, _CHUNK, _COLS), jnp.float32),
        pltpu.SemaphoreType.DMA,
        pltpu.SemaphoreType.DMA,
        pltpu.SemaphoreType.DMA,
        pltpu.SemaphoreType.DMA,
    ],
)
def _sc_copy(x_hbm, o_hbm, buf, in_sem0, in_sem1, out_sem0, out_sem1):
    c = lax.axis_index("c")
    s = lax.axis_index("s")
    w = s * 2 + c
    base = jnp.where(w == _NWORKERS - 1, _LAST_BASE, w * _SHARD)
    base = pl.multiple_of(base, 8)
    in_sems = (in_sem0, in_sem1)
    out_sems = (out_sem0, out_sem1)

    def start_in(j):
        return pltpu.async_copy(
            x_hbm.at[pl.ds(base + _chunk_off(j), _CHUNK), :],
            buf.at[j & 1],
            in_sems[j & 1],
        )

    def start_out(j):
        return pltpu.async_copy(
            buf.at[j & 1],
            o_hbm.at[pl.ds(base + _chunk_off(j), _CHUNK), :],
            out_sems[j & 1],
        )

    ins = {0: start_in(0)}
    outs = {}
    for j in range(_NCH):
        b = j & 1
        ins.pop(j).wait()
        if j + 1 < _NCH:
            if j >= 1:
                outs.pop(j - 1).wait()
            ins[j + 1] = start_in(j + 1)
        if j == 0:
            @pl.when(w == 0)
            def _patch():
                two = jnp.full((16,), 2.0, jnp.float32)
                for i in range(_COLS // 16):
                    buf[b, 1, pl.ds(i * 16, 16)] = two
                    buf[b, 3, pl.ds(i * 16, 16)] = two
        outs[j] = start_out(j)
    outs.pop(_NCH - 2).wait()
    outs.pop(_NCH - 1).wait()


def kernel(x):
    return _sc_copy(x)
